# baseline (speedup 1.0000x reference)
import functools

import jax
import jax.numpy as jnp
from jax import lax
from jax.experimental import pallas as pl
from jax.experimental.pallas import tpu as pltpu
from jax.experimental.pallas import tpu_sc as plsc

STATE = 64
FEAT_C = 64
RO = 4.0
ITER = 3
N, P = 1024, 128
NB = 64

NC, NS, L = 2, 16, 16
NW = NC * NS
PTS_W = (N * P) // NW
CHUNK = 128
NCHUNK = PTS_W // CHUNK


def _gcn_body(feat_ref, poly_ref, cpoly_ref,
              w_in, b_in,
              ws0, wn0, b0, ws1, wn1, b1, ws2, wn2, b2, ws3, wn3, b3,
              w_h, b_h, w_out, b_out,
              pred_ref, npoly_ref, ncpoly_ref):
    nb = feat_ref.shape[1]
    feat = feat_ref[...]
    cp = cpoly_ref[...]
    x = jnp.concatenate([feat, cp * RO], axis=0).reshape(FEAT_C + 2, nb * P)
    h = jax.nn.relu(jnp.dot(w_in[...], x, preferred_element_type=jnp.float32)
                    + b_in[...])
    layers = ((ws0, wn0, b0), (ws1, wn1, b1), (ws2, wn2, b2), (ws3, wn3, b3))
    for ws, wn, b in layers:
        h3 = h.reshape(STATE, nb, P)
        prev = jnp.concatenate([h3[:, :, -1:], h3[:, :, :-1]], axis=2)
        nxt = jnp.concatenate([h3[:, :, 1:], h3[:, :, :1]], axis=2)
        nbr = (prev + nxt).reshape(STATE, nb * P)
        h = jax.nn.relu(jnp.dot(ws[...], h, preferred_element_type=jnp.float32)
                        + jnp.dot(wn[...], nbr, preferred_element_type=jnp.float32)
                        + b[...])
    z = jax.nn.relu(jnp.dot(w_h[...], h, preferred_element_type=jnp.float32)
                    + b_h[...])
    off = jnp.dot(w_out[...], z, preferred_element_type=jnp.float32) + b_out[...]
    pred = poly_ref[...] * RO + off.reshape(2, nb, P)
    pred_ref[...] = pred
    npoly = pred * (1.0 / RO)
    npoly_ref[...] = npoly
    ncpoly_ref[...] = npoly - jnp.min(npoly, axis=2, keepdims=True)


def _gcn_stage(feat, poly, cpoly, p):
    grid = (N // NB,)
    data_spec3 = lambda c: pl.BlockSpec((c, NB, P), lambda i: (0, i, 0))
    full = lambda a: pl.BlockSpec(a.shape, lambda i: (0,) * a.ndim)
    weights = [p['W_in'], p['b_in'].reshape(STATE, 1)]
    for l in range(4):
        weights += [p['Ws%d' % l], p['Wn%d' % l], p['b%d' % l].reshape(STATE, 1)]
    weights += [p['W_h'], p['b_h'].reshape(STATE, 1),
                p['W_out'], p['b_out'].reshape(2, 1)]
    out_shape = [jax.ShapeDtypeStruct((2, N, P), jnp.float32)] * 3
    return pl.pallas_call(
        _gcn_body,
        grid=grid,
        in_specs=[data_spec3(FEAT_C), data_spec3(2), data_spec3(2)]
                 + [full(w) for w in weights],
        out_specs=[data_spec3(2)] * 3,
        out_shape=out_shape,
    )(feat, poly, cpoly, *weights)


def _sc_gather_body(fm_hbm, xs_hbm, ys_hbm, ind_hbm, out_hbm,
                    xs_v, ys_v, ind_v, idx_v, wgt_v, rows_v, outc_v, sem):
    wid = lax.axis_index("s") * NC + lax.axis_index("c")
    tb = wid * PTS_W
    pltpu.sync_copy(xs_hbm.at[pl.ds(tb, PTS_W)], xs_v)
    pltpu.sync_copy(ys_hbm.at[pl.ds(tb, PTS_W)], ys_v)
    pltpu.sync_copy(ind_hbm, ind_v)
    lanes = lax.iota(jnp.int32, L)

    def stage_a(i, carry):
        o = i * L
        x = jnp.clip(xs_v[pl.ds(o, L)], 0.0, 127.0)
        y = jnp.clip(ys_v[pl.ds(o, L)], 0.0, 127.0)
        x0i = x.astype(jnp.int32)
        y0i = y.astype(jnp.int32)
        wx = x - x0i.astype(jnp.float32)
        wy = y - y0i.astype(jnp.float32)
        y1i = jnp.minimum(y0i + 1, 127)
        n_vec = lax.shift_right_logical(tb + o + lanes, 7)
        b = plsc.load_gather(ind_v, [n_vec]) * (128 * 128)
        idx_v[0, pl.ds(o, L)] = b + y0i * 128 + x0i
        idx_v[1, pl.ds(o, L)] = b + y1i * 128 + x0i
        wgt_v[0, pl.ds(o, L)] = (1.0 - wx) * (1.0 - wy)
        wgt_v[1, pl.ds(o, L)] = wx * (1.0 - wy)
        wgt_v[2, pl.ds(o, L)] = (1.0 - wx) * wy
        wgt_v[3, pl.ds(o, L)] = wx * wy
        return carry

    lax.fori_loop(0, PTS_W // L, stage_a, 0)

    def combine_group(g, carry):
        po = carry * CHUNK + g * L
        w00 = wgt_v[0, pl.ds(po, L)]
        w01 = wgt_v[1, pl.ds(po, L)]
        w10 = wgt_v[2, pl.ds(po, L)]
        w11 = wgt_v[3, pl.ds(po, L)]
        r0 = g * L + lanes
        r1 = r0 + CHUNK
        for cc in range(FEAT_C):
            c_lo = jnp.full((L,), cc, jnp.int32)
            c_hi = jnp.full((L,), FEAT_C + cc, jnp.int32)
            acc = (w00 * plsc.load_gather(rows_v, [r0, c_lo])
                   + w01 * plsc.load_gather(rows_v, [r0, c_hi])
                   + w10 * plsc.load_gather(rows_v, [r1, c_lo])
                   + w11 * plsc.load_gather(rows_v, [r1, c_hi]))
            outc_v[cc, pl.ds(g * L, L)] = acc
        return carry

    def chunk_body(c, carry):
        col = c * CHUNK
        copies = [
            pltpu.async_copy(fm_hbm.at[idx_v.at[k, pl.ds(col, CHUNK)]],
                             rows_v.at[pl.ds(k * CHUNK, CHUNK)], sem)
            for k in range(2)
        ]
        for cp in copies:
            cp.wait()
        lax.fori_loop(0, CHUNK // L, combine_group, c)
        pltpu.sync_copy(outc_v, out_hbm.at[:, pl.ds(tb + col, CHUNK)])
        return carry

    lax.fori_loop(0, NCHUNK, chunk_body, 0)


@functools.partial(
    pl.kernel,
    mesh=plsc.VectorSubcoreMesh(core_axis_name="c", subcore_axis_name="s"),
    out_type=jax.ShapeDtypeStruct((FEAT_C, N * P), jnp.float32),
    compiler_params=pltpu.CompilerParams(needs_layout_passes=False),
    scratch_types=[
        pltpu.VMEM((PTS_W,), jnp.float32),
        pltpu.VMEM((PTS_W,), jnp.float32),
        pltpu.VMEM((N,), jnp.int32),
        pltpu.VMEM((2, PTS_W), jnp.int32),
        pltpu.VMEM((4, PTS_W), jnp.float32),
        pltpu.VMEM((2 * CHUNK, 2 * FEAT_C), jnp.float32),
        pltpu.VMEM((FEAT_C, CHUNK), jnp.float32),
        pltpu.SemaphoreType.DMA,
    ],
)
def _sc_gather(*refs):
    _sc_gather_body(*refs)


def kernel(cnn_feature, i_it_ctrs, c_it_ctrs, ind, params):
    B, C, H, W = cnn_feature.shape
    fm_rows = cnn_feature.transpose(0, 2, 3, 1).reshape(B * H * W, C)
    fm2 = jnp.concatenate([fm_rows, jnp.roll(fm_rows, -1, axis=0)], axis=1)
    ind32 = ind.astype(jnp.int32)

    poly = i_it_ctrs.transpose(2, 0, 1)
    cpoly = c_it_ctrs.transpose(2, 0, 1)

    preds = []
    for stage in range(1 + ITER):
        p = params['resgcn'] if stage == 0 else params['resgcn%d' % (stage - 1)]
        xs = poly[0].reshape(N * P)
        ys = poly[1].reshape(N * P)
        feat = _sc_gather(fm2, xs, ys, ind32).reshape(FEAT_C, N, P)
        pred, poly, cpoly = _gcn_stage(feat, poly, cpoly, p)
        preds.append(pred)
    return jnp.stack([pr.transpose(1, 2, 0) for pr in preds])


# speedup vs baseline: 1.8921x; 1.8921x over previous
import functools

import jax
import jax.numpy as jnp
from jax import lax
from jax.experimental import pallas as pl
from jax.experimental.pallas import tpu as pltpu
from jax.experimental.pallas import tpu_sc as plsc

STATE = 64
FEAT_C = 64
RO = 4.0
ITER = 3
N, P = 1024, 128
NB = 32

NC, NS, L = 2, 16, 16
NW = NC * NS
PTS_W = (N * P) // NW
CHUNK = 128
NCHUNK = PTS_W // CHUNK


def _sc_gather_body(fm_hbm, pol_hbm, ind_hbm, out0_hbm, out1_hbm,
                    pol_v, ind_v, idx_v, rows_v, gsem):
    wid = lax.axis_index("s") * NC + lax.axis_index("c")
    tb = wid * PTS_W
    pltpu.sync_copy(pol_hbm.at[pl.ds(2 * tb, 2 * PTS_W)], pol_v)
    pltpu.sync_copy(ind_hbm, ind_v)
    lanes = lax.iota(jnp.int32, L)

    def stage_a(i, carry):
        o = i * L
        pidx = (o + lanes) * 2
        x = jnp.clip(plsc.load_gather(pol_v, [pidx]), 0.0, 127.0)
        y = jnp.clip(plsc.load_gather(pol_v, [pidx + 1]), 0.0, 127.0)
        x0i = x.astype(jnp.int32)
        y0i = y.astype(jnp.int32)
        y1i = jnp.minimum(y0i + 1, 127)
        n_vec = lax.shift_right_logical(tb + o + lanes, 7)
        b = plsc.load_gather(ind_v, [n_vec]) * (128 * 128)
        idx_v[0, pl.ds(o, L)] = b + y0i * 128 + x0i
        idx_v[1, pl.ds(o, L)] = b + y1i * 128 + x0i
        return carry

    lax.fori_loop(0, PTS_W // L, stage_a, 0)

    def fire(c, slot):
        return [
            pltpu.async_copy(fm_hbm.at[idx_v.at[k, pl.ds(c * CHUNK, CHUNK)]],
                             rows_v.at[slot, pl.ds(k * CHUNK, CHUNK)], gsem)
            for k in range(2)
        ]

    pending = fire(0, 0)
    for c in range(NCHUNK):
        nxt = fire(c + 1, (c + 1) % 2) if c + 1 < NCHUNK else []
        for cp in pending:
            cp.wait()
        pending = nxt
        slot = c % 2
        col = tb + c * CHUNK
        pltpu.sync_copy(rows_v.at[slot, pl.ds(0, CHUNK)],
                        out0_hbm.at[pl.ds(col, CHUNK)])
        pltpu.sync_copy(rows_v.at[slot, pl.ds(CHUNK, CHUNK)],
                        out1_hbm.at[pl.ds(col, CHUNK)])


@functools.partial(
    pl.kernel,
    mesh=plsc.VectorSubcoreMesh(core_axis_name="c", subcore_axis_name="s"),
    out_type=[jax.ShapeDtypeStruct((N * P, 2 * FEAT_C), jnp.float32),
              jax.ShapeDtypeStruct((N * P, 2 * FEAT_C), jnp.float32)],
    compiler_params=pltpu.CompilerParams(needs_layout_passes=False),
    scratch_types=[
        pltpu.VMEM((2 * PTS_W,), jnp.float32),
        pltpu.VMEM((N,), jnp.int32),
        pltpu.VMEM((2, PTS_W), jnp.int32),
        pltpu.VMEM((2, 2 * CHUNK, 2 * FEAT_C), jnp.float32),
        pltpu.SemaphoreType.DMA,
    ],
)
def _sc_gather(*refs):
    _sc_gather_body(*refs)


def _mm(a, w):
    return lax.dot_general(a, w, (((1,), (1,)), ((), ())),
                           preferred_element_type=jnp.float32)


def _gcn_body(rows0_ref, rows1_ref, poly_ref, cpoly_ref,
              w_in, b_in,
              ws0, wn0, b0, ws1, wn1, b1, ws2, wn2, b2, ws3, wn3, b3,
              w_h, b_h, w_out, b_out,
              pred_ref, npoly_ref, ncpoly_ref):
    nbp = poly_ref.shape[0]
    pol = poly_ref[...]
    x = jnp.clip(pol[:, 0:1], 0.0, 127.0)
    y = jnp.clip(pol[:, 1:2], 0.0, 127.0)
    wx = x - jnp.floor(x)
    wy = y - jnp.floor(y)
    r0 = rows0_ref[...]
    r1 = rows1_ref[...]
    feat = (r0[:, :FEAT_C] * ((1 - wx) * (1 - wy))
            + r0[:, FEAT_C:] * (wx * (1 - wy))
            + r1[:, :FEAT_C] * ((1 - wx) * wy)
            + r1[:, FEAT_C:] * (wx * wy))
    xin = jnp.concatenate([feat, cpoly_ref[...] * RO], axis=1)
    h = jax.nn.relu(_mm(xin, w_in[...]) + b_in[...])
    layers = ((ws0, wn0, b0), (ws1, wn1, b1), (ws2, wn2, b2), (ws3, wn3, b3))
    for ws, wn, b in layers:
        h3 = h.reshape(nbp // P, P, STATE)
        prev = jnp.concatenate([h3[:, -1:, :], h3[:, :-1, :]], axis=1)
        nxt = jnp.concatenate([h3[:, 1:, :], h3[:, :1, :]], axis=1)
        nbr = (prev + nxt).reshape(nbp, STATE)
        h = jax.nn.relu(_mm(h, ws[...]) + _mm(nbr, wn[...]) + b[...])
    z = jax.nn.relu(_mm(h, w_h[...]) + b_h[...])
    off = _mm(z, w_out[...]) + b_out[...]
    pred = pol * RO + off
    pred_ref[...] = pred
    npoly = pred * (1.0 / RO)
    npoly_ref[...] = npoly
    np3 = npoly.reshape(nbp // P, P, 2)
    ncpoly_ref[...] = (np3 - jnp.min(np3, axis=1, keepdims=True)).reshape(nbp, 2)


def _gcn_stage(rows0, rows1, poly, cpoly, p):
    grid = (N // NB,)
    dspec = lambda c: pl.BlockSpec((NB * P, c), lambda i: (i, 0))
    full = lambda a: pl.BlockSpec(a.shape, lambda i: (0,) * a.ndim)
    weights = [p['W_in'], p['b_in'].reshape(1, STATE)]
    for l in range(4):
        weights += [p['Ws%d' % l], p['Wn%d' % l], p['b%d' % l].reshape(1, STATE)]
    weights += [p['W_h'], p['b_h'].reshape(1, STATE),
                p['W_out'], p['b_out'].reshape(1, 2)]
    out_shape = [jax.ShapeDtypeStruct((N * P, 2), jnp.float32)] * 3
    return pl.pallas_call(
        _gcn_body,
        grid=grid,
        in_specs=[dspec(2 * FEAT_C), dspec(2 * FEAT_C), dspec(2), dspec(2)]
                 + [full(w) for w in weights],
        out_specs=[dspec(2)] * 3,
        out_shape=out_shape,
    )(rows0, rows1, poly, cpoly, *weights)


def kernel(cnn_feature, i_it_ctrs, c_it_ctrs, ind, params):
    B, C, H, W = cnn_feature.shape
    fm_rows = cnn_feature.transpose(0, 2, 3, 1).reshape(B * H * W, C)
    fm2 = jnp.concatenate([fm_rows, jnp.roll(fm_rows, -1, axis=0)], axis=1)
    ind32 = ind.astype(jnp.int32)

    poly = i_it_ctrs.reshape(N * P, 2)
    cpoly = c_it_ctrs.reshape(N * P, 2)

    preds = []
    for stage in range(1 + ITER):
        p = params['resgcn'] if stage == 0 else params['resgcn%d' % (stage - 1)]
        rows0, rows1 = _sc_gather(fm2, poly.reshape(2 * N * P), ind32)
        pred, poly, cpoly = _gcn_stage(rows0, rows1, poly, cpoly, p)
        preds.append(pred)
    return jnp.stack([pr.reshape(N, P, 2) for pr in preds])


# speedup vs baseline: 2.3377x; 1.2355x over previous
import functools

import jax
import jax.numpy as jnp
from jax import lax
from jax.experimental import pallas as pl
from jax.experimental.pallas import tpu as pltpu
from jax.experimental.pallas import tpu_sc as plsc

STATE = 64
FEAT_C = 64
RO = 4.0
ITER = 3
N, P = 1024, 128
NB = 32

NC, NS, L = 2, 16, 16
NW = NC * NS
PTS_W = (N * P) // NW
CHUNK = 128
NCHUNK = PTS_W // CHUNK


def _sc_gather_body(fm_hbm, pol_hbm, ind_hbm, out0_hbm, out1_hbm,
                    pol_v, ind_v, idx_v, rows_v, gsem):
    wid = lax.axis_index("s") * NC + lax.axis_index("c")
    tb = wid * PTS_W
    pltpu.sync_copy(pol_hbm.at[pl.ds(2 * tb, 2 * PTS_W)], pol_v)
    pltpu.sync_copy(ind_hbm, ind_v)
    lanes = lax.iota(jnp.int32, L)

    def stage_a(i, carry):
        o = i * L
        pidx = (o + lanes) * 2
        x = jnp.clip(plsc.load_gather(pol_v, [pidx]), 0.0, 127.0)
        y = jnp.clip(plsc.load_gather(pol_v, [pidx + 1]), 0.0, 127.0)
        x0i = x.astype(jnp.int32)
        y0i = y.astype(jnp.int32)
        y1i = jnp.minimum(y0i + 1, 127)
        n_vec = lax.shift_right_logical(tb + o + lanes, 7)
        b = plsc.load_gather(ind_v, [n_vec]) * (128 * 128)
        idx_v[0, pl.ds(o, L)] = b + y0i * 128 + x0i
        idx_v[1, pl.ds(o, L)] = b + y1i * 128 + x0i
        return carry

    lax.fori_loop(0, PTS_W // L, stage_a, 0)

    def fire(c, slot):
        return [
            pltpu.async_copy(fm_hbm.at[idx_v.at[k, pl.ds(c * CHUNK, CHUNK)]],
                             rows_v.at[slot, pl.ds(k * CHUNK, CHUNK)], gsem)
            for k in range(2)
        ]

    pending = fire(0, 0)
    for c in range(NCHUNK):
        nxt = fire(c + 1, (c + 1) % 2) if c + 1 < NCHUNK else []
        for cp in pending:
            cp.wait()
        pending = nxt
        slot = c % 2
        col = tb + c * CHUNK
        pltpu.sync_copy(rows_v.at[slot, pl.ds(0, CHUNK)],
                        out0_hbm.at[pl.ds(col, CHUNK)])
        pltpu.sync_copy(rows_v.at[slot, pl.ds(CHUNK, CHUNK)],
                        out1_hbm.at[pl.ds(col, CHUNK)])


@functools.cache
def _sc_gather_call():
    return pl.kernel(
        _sc_gather_body,
        mesh=plsc.VectorSubcoreMesh(core_axis_name="c", subcore_axis_name="s"),
        out_type=[jax.ShapeDtypeStruct((N * P, 2 * FEAT_C), jnp.float32),
                  jax.ShapeDtypeStruct((N * P, 2 * FEAT_C), jnp.float32)],
        compiler_params=pltpu.CompilerParams(needs_layout_passes=False),
        scratch_types=[
            pltpu.VMEM((2 * PTS_W,), jnp.float32),
            pltpu.VMEM((N,), jnp.int32),
            pltpu.VMEM((2, PTS_W), jnp.int32),
            pltpu.VMEM((2, 2 * CHUNK, 2 * FEAT_C), jnp.float32),
            pltpu.SemaphoreType.DMA,
        ],
    )


def _sc_gather(fm2, polflat, ind32):
    return _sc_gather_call()(fm2, polflat, ind32)


def _mm(a, w):
    return lax.dot_general(a, w, (((1,), (1,)), ((), ())),
                           preferred_element_type=jnp.float32)


def _gcn_body(rows0_ref, rows1_ref, poly_ref, cpoly_ref,
              sx, sy, w_in2, wc2, b_in,
              ws0, wn0, b0, ws1, wn1, b1, ws2, wn2, b2, ws3, wn3, b3,
              w_h, b_h, w_out, b_out,
              pred_ref, npoly_ref, ncpoly_ref):
    nbp = poly_ref.shape[0]
    pol = poly_ref[...]
    xb = jnp.clip(_mm(pol, sx[...]), 0.0, 127.0)
    yb = jnp.clip(_mm(pol, sy[...]), 0.0, 127.0)
    fx = xb - jnp.floor(xb)
    fy = yb - jnp.floor(yb)
    lane = lax.broadcasted_iota(jnp.int32, (nbp, 2 * FEAT_C), 1)
    wsel = jnp.where(lane < FEAT_C, 1.0 - fx, fx)
    a1 = wsel * fy
    a0 = wsel - a1
    r0 = rows0_ref[...]
    r1 = rows1_ref[...]
    combined = r0 * a0 + r1 * a1
    h = jax.nn.relu(_mm(combined, w_in2[...]) + _mm(cpoly_ref[...], wc2[...])
                    + b_in[...])
    layers = ((ws0, wn0, b0), (ws1, wn1, b1), (ws2, wn2, b2), (ws3, wn3, b3))
    for ws, wn, b in layers:
        h3 = h.reshape(nbp // P, P, STATE)
        prev = jnp.concatenate([h3[:, -1:, :], h3[:, :-1, :]], axis=1)
        nxt = jnp.concatenate([h3[:, 1:, :], h3[:, :1, :]], axis=1)
        nbr = (prev + nxt).reshape(nbp, STATE)
        h = jax.nn.relu(_mm(h, ws[...]) + _mm(nbr, wn[...]) + b[...])
    z = jax.nn.relu(_mm(h, w_h[...]) + b_h[...])
    off = _mm(z, w_out[...]) + b_out[...]
    pred = pol * RO + off
    pred_ref[...] = pred
    npoly = pred * (1.0 / RO)
    npoly_ref[...] = npoly
    np3 = npoly.reshape(nbp // P, P, 2)
    ncpoly_ref[...] = (np3 - jnp.min(np3, axis=1, keepdims=True)).reshape(nbp, 2)


def _gcn_stage(rows0, rows1, poly, cpoly, p):
    grid = (N // NB,)
    dspec = lambda c: pl.BlockSpec((NB * P, c), lambda i: (i, 0))
    full = lambda a: pl.BlockSpec(a.shape, lambda i: (0,) * a.ndim)
    sx = jnp.tile(jnp.array([[1.0, 0.0]], jnp.float32), (2 * FEAT_C, 1))
    sy = jnp.tile(jnp.array([[0.0, 1.0]], jnp.float32), (2 * FEAT_C, 1))
    w_in2 = jnp.concatenate([p['W_in'][:, :FEAT_C]] * 2, axis=1)
    wc2 = p['W_in'][:, FEAT_C:] * RO
    weights = [sx, sy, w_in2, wc2, p['b_in'].reshape(1, STATE)]
    for l in range(4):
        weights += [p['Ws%d' % l], p['Wn%d' % l], p['b%d' % l].reshape(1, STATE)]
    weights += [p['W_h'], p['b_h'].reshape(1, STATE),
                p['W_out'], p['b_out'].reshape(1, 2)]
    out_shape = [jax.ShapeDtypeStruct((N * P, 2), jnp.float32)] * 3
    return pl.pallas_call(
        _gcn_body,
        grid=grid,
        in_specs=[dspec(2 * FEAT_C), dspec(2 * FEAT_C), dspec(2), dspec(2)]
                 + [full(w) for w in weights],
        out_specs=[dspec(2)] * 3,
        out_shape=out_shape,
    )(rows0, rows1, poly, cpoly, *weights)


def kernel(cnn_feature, i_it_ctrs, c_it_ctrs, ind, params):
    B, C, H, W = cnn_feature.shape
    fm_rows = cnn_feature.transpose(0, 2, 3, 1).reshape(B * H * W, C)
    fm2 = jnp.concatenate([fm_rows, jnp.roll(fm_rows, -1, axis=0)], axis=1)
    ind32 = ind.astype(jnp.int32)

    poly = i_it_ctrs.reshape(N * P, 2)
    cpoly = c_it_ctrs.reshape(N * P, 2)

    preds = []
    for stage in range(1 + ITER):
        p = params['resgcn'] if stage == 0 else params['resgcn%d' % (stage - 1)]
        rows0, rows1 = _sc_gather(fm2, poly.reshape(2 * N * P), ind32)
        pred, poly, cpoly = _gcn_stage(rows0, rows1, poly, cpoly, p)
        preds.append(pred)
    return jnp.stack([pr.reshape(N, P, 2) for pr in preds])


# speedup vs baseline: 2.3708x; 1.0141x over previous
import functools

import jax
import jax.numpy as jnp
from jax import lax
from jax.experimental import pallas as pl
from jax.experimental.pallas import tpu as pltpu
from jax.experimental.pallas import tpu_sc as plsc

STATE = 64
FEAT_C = 64
RO = 4.0
ITER = 3
N, P = 1024, 128
NB = 32

NC, NS, L = 2, 16, 16
NW = NC * NS
PTS_W = (N * P) // NW
CHUNK = 128
NCHUNK = PTS_W // CHUNK


@functools.cache
def _sc_gather_call(npts):
    pts_w = npts // NW
    nchunk = pts_w // CHUNK

    def body(fm_hbm, pol_hbm, ind_hbm, out0_hbm, out1_hbm,
             pol_v, ind_v, idx_v, rows_v, gsem):
        wid = lax.axis_index("s") * NC + lax.axis_index("c")
        tb = wid * pts_w
        pltpu.sync_copy(pol_hbm.at[pl.ds(2 * tb, 2 * pts_w)], pol_v)
        pltpu.sync_copy(ind_hbm, ind_v)
        lanes = lax.iota(jnp.int32, L)

        def stage_a(i, carry):
            o = i * L
            pidx = (o + lanes) * 2
            x = jnp.clip(plsc.load_gather(pol_v, [pidx]), 0.0, 127.0)
            y = jnp.clip(plsc.load_gather(pol_v, [pidx + 1]), 0.0, 127.0)
            x0i = x.astype(jnp.int32)
            y0i = y.astype(jnp.int32)
            y1i = jnp.minimum(y0i + 1, 127)
            n_vec = lax.shift_right_logical(tb + o + lanes, 7)
            b = plsc.load_gather(ind_v, [n_vec]) * (128 * 128)
            idx_v[0, pl.ds(o, L)] = b + y0i * 128 + x0i
            idx_v[1, pl.ds(o, L)] = b + y1i * 128 + x0i
            return carry

        lax.fori_loop(0, pts_w // L, stage_a, 0)

        def fire(c, slot):
            return [
                pltpu.async_copy(fm_hbm.at[idx_v.at[k, pl.ds(c * CHUNK, CHUNK)]],
                                 rows_v.at[slot, pl.ds(k * CHUNK, CHUNK)], gsem)
                for k in range(2)
            ]

        pending = fire(0, 0)
        for c in range(nchunk):
            nxt = fire(c + 1, (c + 1) % 2) if c + 1 < nchunk else []
            for cp in pending:
                cp.wait()
            pending = nxt
            slot = c % 2
            col = tb + c * CHUNK
            pltpu.sync_copy(rows_v.at[slot, pl.ds(0, CHUNK)],
                            out0_hbm.at[pl.ds(col, CHUNK)])
            pltpu.sync_copy(rows_v.at[slot, pl.ds(CHUNK, CHUNK)],
                            out1_hbm.at[pl.ds(col, CHUNK)])

    return pl.kernel(
        body,
        mesh=plsc.VectorSubcoreMesh(core_axis_name="c", subcore_axis_name="s"),
        out_type=[jax.ShapeDtypeStruct((npts, 2 * FEAT_C), jnp.float32),
                  jax.ShapeDtypeStruct((npts, 2 * FEAT_C), jnp.float32)],
        compiler_params=pltpu.CompilerParams(needs_layout_passes=False),
        scratch_types=[
            pltpu.VMEM((2 * pts_w,), jnp.float32),
            pltpu.VMEM((npts // P,), jnp.int32),
            pltpu.VMEM((2, pts_w), jnp.int32),
            pltpu.VMEM((2, 2 * CHUNK, 2 * FEAT_C), jnp.float32),
            pltpu.SemaphoreType.DMA,
        ],
    )


def _sc_gather(fm2, polflat, ind32):
    return _sc_gather_call(polflat.shape[0] // 2)(fm2, polflat, ind32)


def _mm(a, w):
    return lax.dot_general(a, w, (((1,), (1,)), ((), ())),
                           preferred_element_type=jnp.float32)


def _gcn_body(rows0_ref, rows1_ref, poly_ref, cpoly_ref,
              sx, sy, m1, s1, w_in2, wc2, b_in,
              ws0, wn0, b0, ws1, wn1, b1, ws2, wn2, b2, ws3, wn3, b3,
              w_h, b_h, w_out, b_out,
              pred_ref, npoly_ref, ncpoly_ref):
    nbp = poly_ref.shape[0]
    pol = poly_ref[...]
    xb = jnp.clip(_mm(pol, sx[...]), 0.0, 127.0)
    yb = jnp.clip(_mm(pol, sy[...]), 0.0, 127.0)
    fx = xb - jnp.floor(xb)
    fy = yb - jnp.floor(yb)
    wsel = m1[...] + s1[...] * fx
    a1 = wsel * fy
    a0 = wsel - a1
    r0 = rows0_ref[...]
    r1 = rows1_ref[...]
    combined = r0 * a0 + r1 * a1
    h = jax.nn.relu(_mm(combined, w_in2[...]) + _mm(cpoly_ref[...], wc2[...])
                    + b_in[...])
    layers = ((ws0, wn0, b0), (ws1, wn1, b1), (ws2, wn2, b2), (ws3, wn3, b3))
    for ws, wn, b in layers:
        h3 = h.reshape(nbp // P, P, STATE)
        prev = jnp.concatenate([h3[:, -1:, :], h3[:, :-1, :]], axis=1)
        nxt = jnp.concatenate([h3[:, 1:, :], h3[:, :1, :]], axis=1)
        nbr = (prev + nxt).reshape(nbp, STATE)
        h = jax.nn.relu(_mm(h, ws[...]) + _mm(nbr, wn[...]) + b[...])
    z = jax.nn.relu(_mm(h, w_h[...]) + b_h[...])
    off = _mm(z, w_out[...]) + b_out[...]
    pred = pol * RO + off
    pred_ref[...] = pred
    npoly = pred * (1.0 / RO)
    npoly_ref[...] = npoly
    np3 = npoly.reshape(nbp // P, P, 2)
    ncpoly_ref[...] = (np3 - jnp.min(np3, axis=1, keepdims=True)).reshape(nbp, 2)


def _gcn_stage(rows0, rows1, poly, cpoly, p):
    npts = poly.shape[0]
    grid = (npts // (NB * P),)
    dspec = lambda c: pl.BlockSpec((NB * P, c), lambda i: (i, 0))
    full = lambda a: pl.BlockSpec(a.shape, lambda i: (0,) * a.ndim)
    sx = jnp.tile(jnp.array([[1.0, 0.0]], jnp.float32), (2 * FEAT_C, 1))
    sy = jnp.tile(jnp.array([[0.0, 1.0]], jnp.float32), (2 * FEAT_C, 1))
    half = jnp.arange(2 * FEAT_C) < FEAT_C
    m1 = jnp.where(half, 1.0, 0.0).reshape(1, 2 * FEAT_C).astype(jnp.float32)
    s1 = jnp.where(half, -1.0, 1.0).reshape(1, 2 * FEAT_C).astype(jnp.float32)
    w_in2 = jnp.concatenate([p['W_in'][:, :FEAT_C]] * 2, axis=1)
    wc2 = p['W_in'][:, FEAT_C:] * RO
    weights = [sx, sy, m1, s1, w_in2, wc2, p['b_in'].reshape(1, STATE)]
    for l in range(4):
        weights += [p['Ws%d' % l], p['Wn%d' % l], p['b%d' % l].reshape(1, STATE)]
    weights += [p['W_h'], p['b_h'].reshape(1, STATE),
                p['W_out'], p['b_out'].reshape(1, 2)]
    out_shape = [jax.ShapeDtypeStruct((npts, 2), jnp.float32)] * 3
    return pl.pallas_call(
        _gcn_body,
        grid=grid,
        in_specs=[dspec(2 * FEAT_C), dspec(2 * FEAT_C), dspec(2), dspec(2)]
                 + [full(w) for w in weights],
        out_specs=[dspec(2)] * 3,
        out_shape=out_shape,
    )(rows0, rows1, poly, cpoly, *weights)


def kernel(cnn_feature, i_it_ctrs, c_it_ctrs, ind, params):
    B, C, H, W = cnn_feature.shape
    fm_rows = cnn_feature.transpose(0, 2, 3, 1).reshape(B * H * W, C)
    fm2 = jnp.concatenate([fm_rows, jnp.roll(fm_rows, -1, axis=0)], axis=1)
    ind32 = ind.astype(jnp.int32)

    nh = N // 2
    preds_h = [[], []]
    for h in range(2):
        sl = slice(h * nh, (h + 1) * nh)
        poly = i_it_ctrs[sl].reshape(nh * P, 2)
        cpoly = c_it_ctrs[sl].reshape(nh * P, 2)
        ind_h = ind32[sl]
        for stage in range(1 + ITER):
            p = (params['resgcn'] if stage == 0
                 else params['resgcn%d' % (stage - 1)])
            rows0, rows1 = _sc_gather(fm2, poly.reshape(2 * nh * P), ind_h)
            pred, poly, cpoly = _gcn_stage(rows0, rows1, poly, cpoly, p)
            preds_h[h].append(pred)
    return jnp.stack([
        jnp.concatenate([preds_h[0][s].reshape(nh, P, 2),
                         preds_h[1][s].reshape(nh, P, 2)], axis=0)
        for s in range(1 + ITER)])


# speedup vs baseline: 2.3731x; 1.0010x over previous
import functools

import jax
import jax.numpy as jnp
from jax import lax
from jax.experimental import pallas as pl
from jax.experimental.pallas import tpu as pltpu
from jax.experimental.pallas import tpu_sc as plsc

STATE = 64
FEAT_C = 64
RO = 4.0
ITER = 3
N, P = 1024, 128
NB = 32

NC, NS, L = 2, 16, 16
NW = NC * NS
PTS_W = (N * P) // NW
CHUNK = 128
NCHUNK = PTS_W // CHUNK


@functools.cache
def _sc_gather_call(npts):
    pts_w = npts // NW
    nchunk = pts_w // CHUNK

    def body(fm_hbm, pol_hbm, ind_hbm, out0_hbm, out1_hbm,
             pol_v, ind_v, idx_v, rows_v, gsem, wsem):
        wid = lax.axis_index("s") * NC + lax.axis_index("c")
        tb = wid * pts_w
        pltpu.sync_copy(pol_hbm.at[pl.ds(2 * tb, 2 * pts_w)], pol_v)
        pltpu.sync_copy(ind_hbm, ind_v)
        lanes = lax.iota(jnp.int32, L)

        def compute_idx(c):
            def group(g, carry):
                o = c * CHUNK + g * L
                pidx = (o + lanes) * 2
                x = jnp.clip(plsc.load_gather(pol_v, [pidx]), 0.0, 127.0)
                y = jnp.clip(plsc.load_gather(pol_v, [pidx + 1]), 0.0, 127.0)
                x0i = x.astype(jnp.int32)
                y0i = y.astype(jnp.int32)
                y1i = jnp.minimum(y0i + 1, 127)
                n_vec = lax.shift_right_logical(tb + o + lanes, 7)
                b = plsc.load_gather(ind_v, [n_vec]) * (128 * 128)
                idx_v[0, pl.ds(o, L)] = b + y0i * 128 + x0i
                idx_v[1, pl.ds(o, L)] = b + y1i * 128 + x0i
                return carry

            lax.fori_loop(0, CHUNK // L, group, 0)

        def fire(c, slot):
            return [
                pltpu.async_copy(fm_hbm.at[idx_v.at[k, pl.ds(c * CHUNK, CHUNK)]],
                                 rows_v.at[slot, pl.ds(k * CHUNK, CHUNK)], gsem)
                for k in range(2)
            ]

        def fire_write(c, slot):
            col = tb + c * CHUNK
            return [
                pltpu.async_copy(rows_v.at[slot, pl.ds(0, CHUNK)],
                                 out0_hbm.at[pl.ds(col, CHUNK)], wsem),
                pltpu.async_copy(rows_v.at[slot, pl.ds(CHUNK, CHUNK)],
                                 out1_hbm.at[pl.ds(col, CHUNK)], wsem),
            ]

        NSLOT = 3
        gathers = {}
        writes = {}
        compute_idx(0)
        gathers[0] = fire(0, 0)
        if nchunk > 1:
            compute_idx(1)
            gathers[1] = fire(1, 1)
        for c in range(nchunk):
            slot = c % NSLOT
            if c + 2 < nchunk:
                compute_idx(c + 2)
                nslot = (c + 2) % NSLOT
                for cp in writes.pop(nslot, []):
                    cp.wait()
                gathers[c + 2] = fire(c + 2, nslot)
            for cp in gathers.pop(c):
                cp.wait()
            writes[slot] = fire_write(c, slot)
        for ws in writes.values():
            for cp in ws:
                cp.wait()

    return pl.kernel(
        body,
        mesh=plsc.VectorSubcoreMesh(core_axis_name="c", subcore_axis_name="s"),
        out_type=[jax.ShapeDtypeStruct((npts, 2 * FEAT_C), jnp.float32),
                  jax.ShapeDtypeStruct((npts, 2 * FEAT_C), jnp.float32)],
        compiler_params=pltpu.CompilerParams(needs_layout_passes=False),
        scratch_types=[
            pltpu.VMEM((2 * pts_w,), jnp.float32),
            pltpu.VMEM((npts // P,), jnp.int32),
            pltpu.VMEM((2, pts_w), jnp.int32),
            pltpu.VMEM((3, 2 * CHUNK, 2 * FEAT_C), jnp.float32),
            pltpu.SemaphoreType.DMA,
            pltpu.SemaphoreType.DMA,
        ],
    )


def _sc_gather(fm2, polflat, ind32):
    return _sc_gather_call(polflat.shape[0] // 2)(fm2, polflat, ind32)


def _mm(a, w):
    return lax.dot_general(a, w, (((1,), (1,)), ((), ())),
                           preferred_element_type=jnp.float32)


def _gcn_body(rows0_ref, rows1_ref, poly_ref, cpoly_ref,
              sx, sy, m1, s1, w_in2, wc2, b_in,
              ws0, wn0, b0, ws1, wn1, b1, ws2, wn2, b2, ws3, wn3, b3,
              w_h, b_h, w_out, b_out,
              pred_ref, npoly_ref, ncpoly_ref):
    nbp = poly_ref.shape[0]
    pol = poly_ref[...]
    xb = jnp.clip(_mm(pol, sx[...]), 0.0, 127.0)
    yb = jnp.clip(_mm(pol, sy[...]), 0.0, 127.0)
    fx = xb - jnp.floor(xb)
    fy = yb - jnp.floor(yb)
    wsel = m1[...] + s1[...] * fx
    a1 = wsel * fy
    a0 = wsel - a1
    r0 = rows0_ref[...]
    r1 = rows1_ref[...]
    combined = r0 * a0 + r1 * a1
    h = jax.nn.relu(_mm(combined, w_in2[...]) + _mm(cpoly_ref[...], wc2[...])
                    + b_in[...])
    layers = ((ws0, wn0, b0), (ws1, wn1, b1), (ws2, wn2, b2), (ws3, wn3, b3))
    for ws, wn, b in layers:
        h3 = h.reshape(nbp // P, P, STATE)
        prev = jnp.concatenate([h3[:, -1:, :], h3[:, :-1, :]], axis=1)
        nxt = jnp.concatenate([h3[:, 1:, :], h3[:, :1, :]], axis=1)
        nbr = (prev + nxt).reshape(nbp, STATE)
        h = jax.nn.relu(_mm(h, ws[...]) + _mm(nbr, wn[...]) + b[...])
    z = jax.nn.relu(_mm(h, w_h[...]) + b_h[...])
    off = _mm(z, w_out[...]) + b_out[...]
    pred = pol * RO + off
    pred_ref[...] = pred
    npoly = pred * (1.0 / RO)
    npoly_ref[...] = npoly
    np3 = npoly.reshape(nbp // P, P, 2)
    ncpoly_ref[...] = (np3 - jnp.min(np3, axis=1, keepdims=True)).reshape(nbp, 2)


def _gcn_stage(rows0, rows1, poly, cpoly, p):
    npts = poly.shape[0]
    grid = (npts // (NB * P),)
    dspec = lambda c: pl.BlockSpec((NB * P, c), lambda i: (i, 0))
    full = lambda a: pl.BlockSpec(a.shape, lambda i: (0,) * a.ndim)
    sx = jnp.tile(jnp.array([[1.0, 0.0]], jnp.float32), (2 * FEAT_C, 1))
    sy = jnp.tile(jnp.array([[0.0, 1.0]], jnp.float32), (2 * FEAT_C, 1))
    half = jnp.arange(2 * FEAT_C) < FEAT_C
    m1 = jnp.where(half, 1.0, 0.0).reshape(1, 2 * FEAT_C).astype(jnp.float32)
    s1 = jnp.where(half, -1.0, 1.0).reshape(1, 2 * FEAT_C).astype(jnp.float32)
    w_in2 = jnp.concatenate([p['W_in'][:, :FEAT_C]] * 2, axis=1)
    wc2 = p['W_in'][:, FEAT_C:] * RO
    weights = [sx, sy, m1, s1, w_in2, wc2, p['b_in'].reshape(1, STATE)]
    for l in range(4):
        weights += [p['Ws%d' % l], p['Wn%d' % l], p['b%d' % l].reshape(1, STATE)]
    weights += [p['W_h'], p['b_h'].reshape(1, STATE),
                p['W_out'], p['b_out'].reshape(1, 2)]
    out_shape = [jax.ShapeDtypeStruct((npts, 2), jnp.float32)] * 3
    return pl.pallas_call(
        _gcn_body,
        grid=grid,
        in_specs=[dspec(2 * FEAT_C), dspec(2 * FEAT_C), dspec(2), dspec(2)]
                 + [full(w) for w in weights],
        out_specs=[dspec(2)] * 3,
        out_shape=out_shape,
    )(rows0, rows1, poly, cpoly, *weights)


def kernel(cnn_feature, i_it_ctrs, c_it_ctrs, ind, params):
    B, C, H, W = cnn_feature.shape
    fm_rows = cnn_feature.transpose(0, 2, 3, 1).reshape(B * H * W, C)
    fm2 = jnp.concatenate([fm_rows, jnp.roll(fm_rows, -1, axis=0)], axis=1)
    ind32 = ind.astype(jnp.int32)

    nh = N // 2
    preds_h = [[], []]
    for h in range(2):
        sl = slice(h * nh, (h + 1) * nh)
        poly = i_it_ctrs[sl].reshape(nh * P, 2)
        cpoly = c_it_ctrs[sl].reshape(nh * P, 2)
        ind_h = ind32[sl]
        for stage in range(1 + ITER):
            p = (params['resgcn'] if stage == 0
                 else params['resgcn%d' % (stage - 1)])
            rows0, rows1 = _sc_gather(fm2, poly.reshape(2 * nh * P), ind_h)
            pred, poly, cpoly = _gcn_stage(rows0, rows1, poly, cpoly, p)
            preds_h[h].append(pred)
    return jnp.stack([
        jnp.concatenate([preds_h[0][s].reshape(nh, P, 2),
                         preds_h[1][s].reshape(nh, P, 2)], axis=0)
        for s in range(1 + ITER)])
